# Initial kernel scaffold; baseline (speedup 1.0000x reference)
import jax
import jax.numpy as jnp
from jax.experimental import pallas as pl


def kernel(position, spin, spin_table, pos_W, pos_b, attn_W, attn_b, down_W, down_b):
    raise NotImplementedError("write your pallas kernel here")


# speedup vs baseline: 2.8012x; 2.8012x over previous
import functools

import jax
import jax.numpy as jnp
from jax import lax
from jax.experimental import pallas as pl
from jax.experimental.pallas import tpu as pltpu
from jax.experimental.pallas import tpu_sc as plsc

_N = 1048576
_H = 64
_NC = 2
_NS = 16
_NW = _NC * _NS
_RPW = _N // _NW
_CH = 4096
_NCHUNK = _RPW // _CH
_L = 16


def _sc_body(pos_hbm, spin_hbm, pmat_hbm, attn_hbm, down_hbm, ab_hbm, db_hbm,
             out_hbm, pos_v, spin_v, out_v, a_v, p_v, d_v, ab_v, db_v):
    wid = lax.axis_index("s") * _NC + lax.axis_index("c")

    pltpu.sync_copy(attn_hbm, a_v)
    pltpu.sync_copy(pmat_hbm, p_v)
    pltpu.sync_copy(down_hbm, d_v)
    pltpu.sync_copy(ab_hbm, ab_v)
    pltpu.sync_copy(db_hbm, db_v)

    iota = lax.iota(jnp.int32, _L)
    zf = jnp.zeros((_L,), jnp.float32)

    colb = [iota * _H + c * (_L * _H) for c in range(4)]

    def _fold(j, carry):
        v0, v1, v2, v3 = carry
        dj = plsc.load_gather(d_v, [jnp.zeros((_L,), jnp.int32) + j])
        c0 = plsc.load_gather(a_v, [colb[0] + j])
        c1 = plsc.load_gather(a_v, [colb[1] + j])
        c2 = plsc.load_gather(a_v, [colb[2] + j])
        c3 = plsc.load_gather(a_v, [colb[3] + j])
        return (v0 + c0 * dj, v1 + c1 * dj, v2 + c2 * dj, v3 + c3 * dj)

    v0, v1, v2, v3 = lax.fori_loop(0, _H, _fold, (zf, zf, zf, zf))

    def _prow_dot(t):
        p0 = p_v[pl.ds(t * _H + 0 * _L, _L)]
        p1 = p_v[pl.ds(t * _H + 1 * _L, _L)]
        p2 = p_v[pl.ds(t * _H + 2 * _L, _L)]
        p3 = p_v[pl.ds(t * _H + 3 * _L, _L)]
        return jnp.sum(p0 * v0 + p1 * v1 + p2 * v2 + p3 * v3)

    w0, w1, w2 = _prow_dot(0), _prow_dot(1), _prow_dot(2)
    s0, s1 = _prow_dot(3), _prow_dot(4)
    cpb = _prow_dot(5)

    dd0 = d_v[pl.ds(0 * _L, _L)]
    dd1 = d_v[pl.ds(1 * _L, _L)]
    dd2 = d_v[pl.ds(2 * _L, _L)]
    dd3 = d_v[pl.ds(3 * _L, _L)]
    ab0 = ab_v[pl.ds(0 * _L, _L)]
    ab1 = ab_v[pl.ds(1 * _L, _L)]
    ab2 = ab_v[pl.ds(2 * _L, _L)]
    ab3 = ab_v[pl.ds(3 * _L, _L)]
    cab = jnp.sum(ab0 * dd0 + ab1 * dd1 + ab2 * dd2 + ab3 * dd3)

    base_c = db_v[pl.ds(0, _L)] + (cpb + cab)
    off0 = base_c + s0
    off1 = base_c + s1
    w0v = zf + w0
    w1v = zf + w1
    w2v = zf + w2

    iota3 = iota * 3

    for ch in range(_NCHUNK):
        base = wid * _RPW + ch * _CH
        pltpu.sync_copy(pos_hbm.at[pl.ds(base * 3, _CH * 3)], pos_v)
        pltpu.sync_copy(spin_hbm.at[pl.ds(base, _CH)], spin_v)

        def _step(j, carry):
            b3 = j * (3 * _L)
            xs = plsc.load_gather(pos_v, [b3 + iota3])
            ys = plsc.load_gather(pos_v, [b3 + iota3 + 1])
            zs = plsc.load_gather(pos_v, [b3 + iota3 + 2])
            sv = spin_v[pl.ds(j * _L, _L)]
            res = (xs * w0v + ys * w1v + zs * w2v
                   + jnp.where(sv > 0.0, off1, off0))
            out_v[pl.ds(j * _L, _L)] = res
            return carry

        lax.fori_loop(0, _CH // _L, _step, 0)
        pltpu.sync_copy(out_v, out_hbm.at[pl.ds(base, _CH)])


@jax.jit
def _chi_sc(posf, spinf, pmat, attnf, downf, ab, db64):
    mesh = plsc.VectorSubcoreMesh(core_axis_name="c", subcore_axis_name="s",
                                  num_cores=_NC, num_subcores=_NS)
    return pl.kernel(
        _sc_body,
        out_type=jax.ShapeDtypeStruct((_N,), jnp.float32),
        mesh=mesh,
        compiler_params=pltpu.CompilerParams(needs_layout_passes=False),
        scratch_types=[
            pltpu.VMEM((_CH * 3,), jnp.float32),
            pltpu.VMEM((_CH,), jnp.float32),
            pltpu.VMEM((_CH,), jnp.float32),
            pltpu.VMEM((_H * _H,), jnp.float32),
            pltpu.VMEM((6 * _H,), jnp.float32),
            pltpu.VMEM((_H,), jnp.float32),
            pltpu.VMEM((_H,), jnp.float32),
            pltpu.VMEM((_H,), jnp.float32),
        ],
    )(posf, spinf, pmat, attnf, downf, ab, db64)


def kernel(position, spin, spin_table, pos_W, pos_b, attn_W, attn_b, down_W,
           down_b):
    posf = position.reshape(-1)
    spinf = spin.reshape(-1)
    pmat = jnp.concatenate(
        [pos_W, spin_table, pos_b[None, :]], axis=0).reshape(-1)
    attnf = attn_W.reshape(-1)
    downf = down_W.reshape(-1)
    db64 = jnp.broadcast_to(down_b, (_H,))
    out = _chi_sc(posf, spinf, pmat, attnf, downf, attn_b, db64)
    return out.reshape(_N, 1)


# speedup vs baseline: 3.1871x; 1.1378x over previous
import functools

import jax
import jax.numpy as jnp
from jax import lax
from jax.experimental import pallas as pl
from jax.experimental.pallas import tpu as pltpu

_N = 1048576
_H = 64
_B = 4096
_G = _N // _B


def _dot_bf16(a, b):
    return lax.dot_general(
        a.astype(jnp.bfloat16), b.astype(jnp.bfloat16),
        (((1,), (0,)), ((), ())),
        preferred_element_type=jnp.float32)


def _body(pos_ref, spin_ref, st_ref, pw_ref, pb_ref, aw_ref, ab_ref, dw_ref,
          db_ref, out_ref):
    p = pos_ref[...]
    pe = _dot_bf16(p, pw_ref[...])
    pe = pe + pb_ref[...]
    st = st_ref[...]
    ind = spin_ref[...] > 0.0
    comb = pe + jnp.where(ind, st[1:2, :], st[0:1, :])
    att = _dot_bf16(comb, aw_ref[...]) + ab_ref[...]
    out_ref[...] = _dot_bf16(att, dw_ref[...]) + db_ref[...]


@jax.jit
def _chi_tc(position, spin, spin_table, pos_W, pos_b2, attn_W, attn_b2,
            down_W, down_b2):
    return pl.pallas_call(
        _body,
        grid=(_G,),
        in_specs=[
            pl.BlockSpec((_B, 3), lambda i: (i, 0)),
            pl.BlockSpec((_B, 1), lambda i: (i, 0)),
            pl.BlockSpec((2, _H), lambda i: (0, 0)),
            pl.BlockSpec((3, _H), lambda i: (0, 0)),
            pl.BlockSpec((1, _H), lambda i: (0, 0)),
            pl.BlockSpec((_H, _H), lambda i: (0, 0)),
            pl.BlockSpec((1, _H), lambda i: (0, 0)),
            pl.BlockSpec((_H, 1), lambda i: (0, 0)),
            pl.BlockSpec((1, 1), lambda i: (0, 0)),
        ],
        out_specs=pl.BlockSpec((_B, 1), lambda i: (i, 0)),
        out_shape=jax.ShapeDtypeStruct((_N, 1), jnp.float32),
        compiler_params=pltpu.CompilerParams(
            dimension_semantics=("arbitrary",)),
    )(position, spin, spin_table, pos_W, pos_b2, attn_W, attn_b2, down_W,
      down_b2)


def kernel(position, spin, spin_table, pos_W, pos_b, attn_W, attn_b, down_W,
           down_b):
    return _chi_tc(position, spin, spin_table, pos_W, pos_b[None, :], attn_W,
                   attn_b[None, :], down_W, down_b[None, :])
